# baseline (speedup 1.0000x reference)
import jax
import jax.numpy as jnp
from jax.experimental import pallas as pl
from jax.experimental.pallas import tpu as pltpu

EMBED_DIM = 4096
NUM_EXPERTS = 64
NUM_CLASSES = 2
B = 4
SEQ = 2048
EXPERTS_PER_CLASS = NUM_EXPERTS // NUM_CLASSES
M_BLK = 512


def _router_kernel(cls_ref, x_ref, wt_ref, b_ref, out_ref):
    m = pl.program_id(0)
    batch = (m * M_BLK) // SEQ
    cls = cls_ref[batch]
    xb = x_ref[...]
    wt = wt_ref[...]
    logits = jnp.dot(xb, wt, preferred_element_type=jnp.float32)
    logits = logits + b_ref[...]
    e = jax.lax.broadcasted_iota(jnp.int32, logits.shape, 1)
    in_class = (e // EXPERTS_PER_CLASS) == cls
    logits = jnp.where(in_class, logits, -jnp.inf)
    mx = jnp.max(logits, axis=-1, keepdims=True)
    ex = jnp.exp(logits - mx)
    out_ref[...] = ex / jnp.sum(ex, axis=-1, keepdims=True)


def kernel(x, class_label, W, b):
    x2d = x.reshape(B * SEQ, EMBED_DIM)
    wt = W.T
    b2d = b.reshape(1, NUM_EXPERTS)
    cls_i32 = class_label.astype(jnp.int32)
    grid = (B * SEQ) // M_BLK
    out = pl.pallas_call(
        _router_kernel,
        grid_spec=pltpu.PrefetchScalarGridSpec(
            num_scalar_prefetch=1,
            grid=(grid,),
            in_specs=[
                pl.BlockSpec((M_BLK, EMBED_DIM), lambda m, c: (m, 0)),
                pl.BlockSpec((EMBED_DIM, NUM_EXPERTS), lambda m, c: (0, 0)),
                pl.BlockSpec((1, NUM_EXPERTS), lambda m, c: (0, 0)),
            ],
            out_specs=pl.BlockSpec((M_BLK, NUM_EXPERTS), lambda m, c: (m, 0)),
        ),
        out_shape=jax.ShapeDtypeStruct((B * SEQ, NUM_EXPERTS), jnp.float32),
        compiler_params=pltpu.CompilerParams(
            dimension_semantics=("arbitrary",),
        ),
    )(cls_i32, x2d, wt, b2d)
    return out.reshape(B, SEQ, NUM_EXPERTS)


# speedup vs baseline: 1.0101x; 1.0101x over previous
import jax
import jax.numpy as jnp
from jax.experimental import pallas as pl
from jax.experimental.pallas import tpu as pltpu

EMBED_DIM = 4096
NUM_EXPERTS = 64
NUM_CLASSES = 2
B = 4
SEQ = 2048
EXPERTS_PER_CLASS = NUM_EXPERTS // NUM_CLASSES
M_BLK = 512


def _router_kernel(cls_ref, x_ref, wt_ref, b_ref, out_ref):
    m = pl.program_id(0)
    batch = (m * M_BLK) // SEQ
    cls = cls_ref[batch]
    xb = x_ref[...].astype(jnp.bfloat16)
    wt = wt_ref[...]
    logits = jnp.dot(xb, wt, preferred_element_type=jnp.float32)
    logits = logits + b_ref[...]
    e = jax.lax.broadcasted_iota(jnp.int32, logits.shape, 1)
    in_class = (e // EXPERTS_PER_CLASS) == cls
    logits = jnp.where(in_class, logits, -jnp.inf)
    mx = jnp.max(logits, axis=-1, keepdims=True)
    ex = jnp.exp(logits - mx)
    out_ref[...] = ex / jnp.sum(ex, axis=-1, keepdims=True)


def kernel(x, class_label, W, b):
    x2d = x.reshape(B * SEQ, EMBED_DIM)
    wt = W.T.astype(jnp.bfloat16)
    b2d = b.reshape(1, NUM_EXPERTS)
    cls_i32 = class_label.astype(jnp.int32)
    grid = (B * SEQ) // M_BLK
    out = pl.pallas_call(
        _router_kernel,
        grid_spec=pltpu.PrefetchScalarGridSpec(
            num_scalar_prefetch=1,
            grid=(grid,),
            in_specs=[
                pl.BlockSpec((M_BLK, EMBED_DIM), lambda m, c: (m, 0)),
                pl.BlockSpec((EMBED_DIM, NUM_EXPERTS), lambda m, c: (0, 0)),
                pl.BlockSpec((1, NUM_EXPERTS), lambda m, c: (0, 0)),
            ],
            out_specs=pl.BlockSpec((M_BLK, NUM_EXPERTS), lambda m, c: (m, 0)),
        ),
        out_shape=jax.ShapeDtypeStruct((B * SEQ, NUM_EXPERTS), jnp.float32),
        compiler_params=pltpu.CompilerParams(
            dimension_semantics=("arbitrary",),
        ),
    )(cls_i32, x2d, wt, b2d)
    return out.reshape(B, SEQ, NUM_EXPERTS)


# speedup vs baseline: 1.0324x; 1.0221x over previous
import jax
import jax.numpy as jnp
from jax.experimental import pallas as pl
from jax.experimental.pallas import tpu as pltpu

EMBED_DIM = 4096
NUM_EXPERTS = 64
NUM_CLASSES = 2
B = 4
SEQ = 2048
EXPERTS_PER_CLASS = NUM_EXPERTS // NUM_CLASSES
M_BLK = 512


def _router_kernel(cls_ref, x_ref, wt_ref, b_ref, out_ref):
    m = pl.program_id(0)
    batch = (m * M_BLK) // SEQ
    cls = cls_ref[batch]
    xb = x_ref[...].astype(jnp.bfloat16)
    wt = wt_ref[...]
    logits = jnp.dot(xb, wt, preferred_element_type=jnp.float32)
    logits = logits + b_ref[...]
    e = jax.lax.broadcasted_iota(jnp.int32, logits.shape, 1)
    in_class = (e // EXPERTS_PER_CLASS) == cls
    logits = jnp.where(in_class, logits, -jnp.inf)
    mx = jnp.max(logits, axis=-1, keepdims=True)
    ex = jnp.exp(logits - mx)
    out_ref[...] = ex / jnp.sum(ex, axis=-1, keepdims=True)


def kernel(x, class_label, W, b):
    x2d = x.reshape(B * SEQ, EMBED_DIM)
    wt = W.T.astype(jnp.bfloat16)
    b2d = b.reshape(1, NUM_EXPERTS)
    cls_i32 = class_label.astype(jnp.int32)
    grid = (B * SEQ) // M_BLK
    out = pl.pallas_call(
        _router_kernel,
        grid_spec=pltpu.PrefetchScalarGridSpec(
            num_scalar_prefetch=1,
            grid=(grid,),
            in_specs=[
                pl.BlockSpec((M_BLK, EMBED_DIM), lambda m, c: (m, 0)),
                pl.BlockSpec((EMBED_DIM, NUM_EXPERTS), lambda m, c: (0, 0)),
                pl.BlockSpec((1, NUM_EXPERTS), lambda m, c: (0, 0)),
            ],
            out_specs=pl.BlockSpec((M_BLK, NUM_EXPERTS), lambda m, c: (m, 0)),
        ),
        out_shape=jax.ShapeDtypeStruct((B * SEQ, NUM_EXPERTS), jnp.float32),
        compiler_params=pltpu.CompilerParams(
            dimension_semantics=("parallel",),
        ),
    )(cls_i32, x2d, wt, b2d)
    return out.reshape(B, SEQ, NUM_EXPERTS)
